# Initial kernel scaffold; baseline (speedup 1.0000x reference)
import jax
import jax.numpy as jnp
from jax.experimental import pallas as pl


def kernel(categorical_x, numerical_x, emb_table, W, b):
    raise NotImplementedError("write your pallas kernel here")


# speedup vs baseline: 1.8688x; 1.8688x over previous
import functools

import jax
import jax.numpy as jnp
from jax import lax
from jax.experimental import pallas as pl
from jax.experimental.pallas import tpu as pltpu
from jax.experimental.pallas import tpu_sc as plsc

B = 16384
NF = 16
NN = 63
NNP = 64
EMB = 128
NR = NF + 1

_info = plsc.get_sparse_core_info()
NC = _info.num_cores
NS = _info.num_subcores
NW = NC * NS
SPW = B // NW
CH = 16
NCHUNK = SPW // CH
G = 4
NJ = EMB // 16

_mesh = plsc.VectorSubcoreMesh(core_axis_name="c", subcore_axis_name="s")


@functools.partial(
    pl.kernel,
    out_type=jax.ShapeDtypeStruct((B, NR, EMB), jnp.float32),
    mesh=_mesh,
    scratch_types=[
        pltpu.VMEM((CH, NF), jnp.int32),
        pltpu.VMEM((CH, NNP), jnp.float32),
        pltpu.VMEM((CH, NR, EMB), jnp.float32),
        pltpu.VMEM((NNP, EMB), jnp.float32),
        pltpu.VMEM((EMB,), jnp.float32),
        pltpu.SemaphoreType.DMA,
    ],
)
def _emb_kernel(cat_hbm, x_hbm, tbl_hbm, wt_hbm, b_hbm, out_hbm,
                idx_v, x_v, obuf, wt_v, b_v, sem):
    wid = lax.axis_index("c") * NS + lax.axis_index("s")
    base0 = wid * SPW

    pltpu.sync_copy(wt_hbm, wt_v)
    pltpu.sync_copy(b_hbm, b_v)

    def chunk_body(ci, carry):
        base = base0 + ci * CH
        pltpu.sync_copy(cat_hbm.at[pl.ds(base, CH)], idx_v)
        pltpu.sync_copy(x_hbm.at[pl.ds(base, CH)], x_v)

        copies = [
            pltpu.async_copy(tbl_hbm.at[idx_v.at[i]],
                             obuf.at[i, pl.ds(0, NF)], sem)
            for i in range(CH)
        ]

        def lin_body(g, carry2):
            s0 = g * G
            acc = [[b_v[pl.ds(16 * j, 16)] for j in range(NJ)]
                   for _ in range(G)]
            xrows = [[x_v[s0 + gi, pl.ds(16 * t, 16)] for t in range(NNP // 16)]
                     for gi in range(G)]
            for k in range(NN):
                w_k = [wt_v[k, pl.ds(16 * j, 16)] for j in range(NJ)]
                for gi in range(G):
                    xv = xrows[gi][k // 16][k % 16]
                    for j in range(NJ):
                        acc[gi][j] = acc[gi][j] + xv * w_k[j]
            for gi in range(G):
                for j in range(NJ):
                    obuf[s0 + gi, NF, pl.ds(16 * j, 16)] = acc[gi][j]
            return carry2

        lax.fori_loop(0, CH // G, lin_body, 0)

        for c in copies:
            c.wait()

        pltpu.sync_copy(obuf, out_hbm.at[pl.ds(base, CH)])
        return carry

    lax.fori_loop(0, NCHUNK, chunk_body, 0)


def kernel(categorical_x, numerical_x, emb_table, W, b):
    cat = categorical_x.astype(jnp.int32)
    xp = jnp.pad(numerical_x, ((0, 0), (0, NNP - NN)))
    wt = jnp.pad(W.T, ((0, NNP - NN), (0, 0)))
    out3 = _emb_kernel(cat, xp, emb_table, wt, b)
    return out3.reshape(B, NR * EMB)


# speedup vs baseline: 3.0395x; 1.6264x over previous
import functools

import jax
import jax.numpy as jnp
from jax import lax
from jax.experimental import pallas as pl
from jax.experimental.pallas import tpu as pltpu
from jax.experimental.pallas import tpu_sc as plsc

B = 16384
NF = 16
NN = 63
NNP = 64
EMB = 128
NR = NF + 1

_info = plsc.get_sparse_core_info()
NC = _info.num_cores
NS = _info.num_subcores
NW = NC * NS
SPW = B // NW
CH = 16
CB = CH // 8
NCHUNK = SPW // CH
G = 4
NJ = EMB // 16

_mesh = plsc.VectorSubcoreMesh(core_axis_name="c", subcore_axis_name="s")


@functools.partial(
    pl.kernel,
    out_type=jax.ShapeDtypeStruct((B // 8, NR * 8, EMB), jnp.float32),
    mesh=_mesh,
    scratch_types=[
        pltpu.VMEM((SPW // 8, NF * 8), jnp.int32),
        pltpu.VMEM((2, CH, NNP), jnp.float32),
        pltpu.VMEM((2, CB, NR * 8, EMB), jnp.float32),
        pltpu.VMEM((NNP, EMB), jnp.float32),
        pltpu.VMEM((EMB,), jnp.float32),
        pltpu.SemaphoreType.DMA,
        pltpu.SemaphoreType.DMA,
        pltpu.SemaphoreType.DMA,
    ],
)
def _emb_kernel(catp_hbm, x_hbm, tbl_hbm, wt_hbm, b_hbm, out_hbm,
                idx_v, x_v, obuf, wt_v, b_v,
                x_sem, g_sem, out_sem):
    wid = lax.axis_index("c") * NS + lax.axis_index("s")
    base0 = pl.multiple_of(wid * SPW, 8)
    blk0 = pl.multiple_of(wid * (SPW // 8), 8)

    pltpu.sync_copy(wt_hbm, wt_v)
    pltpu.sync_copy(b_hbm, b_v)
    pltpu.sync_copy(catp_hbm.at[pl.ds(pl.multiple_of(blk0, 8), SPW // 8)],
                    idx_v)

    def in_copies(c, p):
        return (
            pltpu.make_async_copy(
                x_hbm.at[pl.ds(pl.multiple_of(base0 + c * CH, 8), CH)],
                x_v.at[p], x_sem),
        )

    def out_copy(c, p):
        return pltpu.make_async_copy(
            obuf.at[p], out_hbm.at[pl.ds(blk0 + c * CB, CB)], out_sem)

    def gather_copy(ci, p, kb):
        return pltpu.make_async_copy(
            tbl_hbm.at[idx_v.at[ci * CB + kb]],
            obuf.at[p, kb, pl.ds(0, NF * 8)], g_sem)

    for cpy in in_copies(0, 0):
        cpy.start()

    def chunk_body(ci, carry):
        p = lax.rem(ci, 2)
        pn = lax.rem(ci + 1, 2)

        @pl.when(ci >= 2)
        def _():
            out_copy(ci - 2, p).wait()

        for cpy in in_copies(ci, p):
            cpy.wait()

        for kb in range(CB):
            gather_copy(ci, p, kb).start()

        cn = jnp.minimum(ci + 1, NCHUNK - 1)
        for cpy in in_copies(cn, pn):
            cpy.start()

        def lin_group(g, carry2):
            s0 = g * G
            kb = s0 // 8
            r0 = NF * 8 + lax.rem(s0, 8)
            acc = [[b_v[pl.ds(16 * j, 16)] for j in range(NJ)]
                   for _ in range(G)]
            xrows = [[x_v[p, s0 + gi, pl.ds(16 * t, 16)]
                      for t in range(NNP // 16)] for gi in range(G)]
            for k in range(NN):
                w_k = [wt_v[k, pl.ds(16 * j, 16)] for j in range(NJ)]
                for gi in range(G):
                    xv = xrows[gi][k // 16][k % 16]
                    for j in range(NJ):
                        acc[gi][j] = acc[gi][j] + xv * w_k[j]
            for gi in range(G):
                for j in range(NJ):
                    obuf[p, kb, r0 + gi, pl.ds(16 * j, 16)] = acc[gi][j]
            return carry2

        lax.fori_loop(0, CH // G, lin_group, 0)

        for kb in range(CB):
            gather_copy(ci, p, kb).wait()
        out_copy(ci, p).start()
        return carry

    lax.fori_loop(0, NCHUNK, chunk_body, 0)

    for cpy in in_copies(NCHUNK - 1, lax.rem(NCHUNK, 2)):
        cpy.wait()
    out_copy(NCHUNK - 2, 0).wait()
    out_copy(NCHUNK - 1, 1).wait()


def kernel(categorical_x, numerical_x, emb_table, W, b):
    cat = categorical_x.astype(jnp.int32)
    catp = cat.reshape(B // 8, 8, NF).transpose(0, 2, 1).reshape(B // 8, NF * 8)
    xp = jnp.pad(numerical_x, ((0, 0), (0, NNP - NN)))
    wt = jnp.pad(W.T, ((0, NNP - NN), (0, 0)))
    out4 = _emb_kernel(catp, xp, emb_table, wt, b)
    out4 = out4.reshape(B // 8, NR, 8, EMB)
    return out4.transpose(0, 2, 1, 3).reshape(B, NR * EMB)


# speedup vs baseline: 7.2350x; 2.3803x over previous
import functools

import jax
import jax.numpy as jnp
from jax import lax
from jax.experimental import pallas as pl
from jax.experimental.pallas import tpu as pltpu
from jax.experimental.pallas import tpu_sc as plsc

B = 16384
NF = 16
NN = 63
NNP = 64
EMB = 128
NR = NF + 1

_info = plsc.get_sparse_core_info()
NC = _info.num_cores
NS = _info.num_subcores
NW = NC * NS
SPW = B // NW
CH = 16
CB = CH // 8
NCHUNK = SPW // CH
G = 4
NJ = EMB // 16

_mesh = plsc.VectorSubcoreMesh(core_axis_name="c", subcore_axis_name="s")


@functools.partial(
    pl.kernel,
    out_type=jax.ShapeDtypeStruct((B // 8, NR * 8, EMB), jnp.float32),
    mesh=_mesh,
    scratch_types=[
        pltpu.VMEM((SPW // 8, NF * 8), jnp.int32),
        pltpu.VMEM((2, CH, NNP), jnp.float32),
        pltpu.VMEM((2, CB, NR * 8, EMB), jnp.float32),
        pltpu.VMEM((NNP, EMB), jnp.float32),
        pltpu.VMEM((EMB,), jnp.float32),
        pltpu.VMEM_SHARED((128, EMB), jnp.float32),
        pltpu.SemaphoreType.DMA,
        pltpu.SemaphoreType.DMA,
        pltpu.SemaphoreType.DMA,
    ],
)
def _emb_kernel(catp_hbm, x_hbm, tbl_hbm, wt_hbm, b_hbm, out_hbm,
                idx_v, x_v, obuf, wt_v, b_v, tbl_sh,
                x_sem, g_sem, out_sem):
    wid = lax.axis_index("c") * NS + lax.axis_index("s")
    base0 = pl.multiple_of(wid * SPW, 8)
    blk0 = pl.multiple_of(wid * (SPW // 8), 8)

    @pl.when(lax.axis_index("s") == 0)
    def _():
        pltpu.sync_copy(tbl_hbm, tbl_sh.at[pl.ds(0, 127)])

    pltpu.sync_copy(wt_hbm, wt_v)
    pltpu.sync_copy(b_hbm, b_v)
    pltpu.sync_copy(catp_hbm.at[pl.ds(pl.multiple_of(blk0, 8), SPW // 8)],
                    idx_v)
    plsc.subcore_barrier()

    def in_copies(c, p):
        return (
            pltpu.make_async_copy(
                x_hbm.at[pl.ds(pl.multiple_of(base0 + c * CH, 8), CH)],
                x_v.at[p], x_sem),
        )

    def out_copy(c, p):
        return pltpu.make_async_copy(
            obuf.at[p], out_hbm.at[pl.ds(blk0 + c * CB, CB)], out_sem)

    def gather_copy(ci, p, kb):
        return pltpu.make_async_copy(
            tbl_sh.at[idx_v.at[ci * CB + kb]],
            obuf.at[p, kb, pl.ds(0, NF * 8)], g_sem)

    for cpy in in_copies(0, 0):
        cpy.start()

    def chunk_body(ci, carry):
        p = lax.rem(ci, 2)
        pn = lax.rem(ci + 1, 2)

        @pl.when(ci >= 2)
        def _():
            out_copy(ci - 2, p).wait()

        for cpy in in_copies(ci, p):
            cpy.wait()

        for kb in range(CB):
            gather_copy(ci, p, kb).start()

        cn = jnp.minimum(ci + 1, NCHUNK - 1)
        for cpy in in_copies(cn, pn):
            cpy.start()

        def lin_group(g, carry2):
            s0 = g * G
            kb = s0 // 8
            r0 = NF * 8 + lax.rem(s0, 8)
            acc = [[b_v[pl.ds(16 * j, 16)] for j in range(NJ)]
                   for _ in range(G)]
            xrows = [[x_v[p, s0 + gi, pl.ds(16 * t, 16)]
                      for t in range(NNP // 16)] for gi in range(G)]
            for k in range(NN):
                w_k = [wt_v[k, pl.ds(16 * j, 16)] for j in range(NJ)]
                for gi in range(G):
                    xv = xrows[gi][k // 16][k % 16]
                    for j in range(NJ):
                        acc[gi][j] = acc[gi][j] + xv * w_k[j]
            for gi in range(G):
                for j in range(NJ):
                    obuf[p, kb, r0 + gi, pl.ds(16 * j, 16)] = acc[gi][j]
            return carry2

        lax.fori_loop(0, CH // G, lin_group, 0)

        for kb in range(CB):
            gather_copy(ci, p, kb).wait()
        out_copy(ci, p).start()
        return carry

    lax.fori_loop(0, NCHUNK, chunk_body, 0)

    for cpy in in_copies(NCHUNK - 1, lax.rem(NCHUNK, 2)):
        cpy.wait()
    out_copy(NCHUNK - 2, 0).wait()
    out_copy(NCHUNK - 1, 1).wait()


def kernel(categorical_x, numerical_x, emb_table, W, b):
    cat = categorical_x.astype(jnp.int32)
    catp = cat.reshape(B // 8, 8, NF).transpose(0, 2, 1).reshape(B // 8, NF * 8)
    xp = jnp.pad(numerical_x, ((0, 0), (0, NNP - NN)))
    wt = jnp.pad(W.T, ((0, NNP - NN), (0, 0)))
    out4 = _emb_kernel(catp, xp, emb_table, wt, b)
    out4 = out4.reshape(B // 8, NR, 8, EMB)
    return out4.transpose(0, 2, 1, 3).reshape(B, NR * EMB)


# speedup vs baseline: 11.3592x; 1.5700x over previous
import functools

import jax
import jax.numpy as jnp
from jax import lax
from jax.experimental import pallas as pl
from jax.experimental.pallas import tpu as pltpu
from jax.experimental.pallas import tpu_sc as plsc

B = 16384
NF = 16
NN = 63
NNP = 64
EMB = 128
NR = NF + 1

_info = plsc.get_sparse_core_info()
NC = _info.num_cores
NS = _info.num_subcores
NW = NC * NS
SPW = B // NW
CH = 16
CB = CH // 8
NCHUNK = SPW // CH

TB = 1024

_mesh = plsc.VectorSubcoreMesh(core_axis_name="c", subcore_axis_name="s")


def _lin_body(x_ref, w_ref, b_ref, o_ref):
    o_ref[...] = (
        jnp.dot(x_ref[...], w_ref[...], preferred_element_type=jnp.float32)
        + b_ref[...]
    )


_lin_kernel = pl.pallas_call(
    _lin_body,
    out_shape=jax.ShapeDtypeStruct((B, EMB), jnp.float32),
    grid=(B // TB,),
    in_specs=[
        pl.BlockSpec((TB, NNP), lambda i: (i, 0)),
        pl.BlockSpec((NNP, EMB), lambda i: (0, 0)),
        pl.BlockSpec((1, EMB), lambda i: (0, 0)),
    ],
    out_specs=pl.BlockSpec((TB, EMB), lambda i: (i, 0)),
)


@functools.partial(
    pl.kernel,
    out_type=jax.ShapeDtypeStruct((B // 8, NR * 8, EMB), jnp.float32),
    mesh=_mesh,
    scratch_types=[
        pltpu.VMEM((SPW // 8, NF * 8), jnp.int32),
        pltpu.VMEM((2, CB, NR * 8, EMB), jnp.float32),
        pltpu.VMEM_SHARED((128, EMB), jnp.float32),
        pltpu.SemaphoreType.DMA,
        pltpu.SemaphoreType.DMA,
        pltpu.SemaphoreType.DMA,
    ],
)
def _emb_kernel(catp_hbm, nemb_hbm, tbl_hbm, out_hbm,
                idx_v, obuf, tbl_sh, g_sem, n_sem, out_sem):
    wid = lax.axis_index("c") * NS + lax.axis_index("s")
    base0 = pl.multiple_of(wid * SPW, 8)
    blk0 = pl.multiple_of(wid * (SPW // 8), 8)

    @pl.when(lax.axis_index("s") == 0)
    def _():
        pltpu.sync_copy(tbl_hbm, tbl_sh.at[pl.ds(0, 127)])

    pltpu.sync_copy(catp_hbm.at[pl.ds(blk0, SPW // 8)], idx_v)
    plsc.subcore_barrier()

    def gather_copy(ci, p, kb):
        return pltpu.make_async_copy(
            tbl_sh.at[idx_v.at[ci * CB + kb]],
            obuf.at[p, kb, pl.ds(0, NF * 8)], g_sem)

    def nemb_copy(ci, p, kb):
        return pltpu.make_async_copy(
            nemb_hbm.at[pl.ds(pl.multiple_of(base0 + ci * CH + kb * 8, 8), 8)],
            obuf.at[p, kb, pl.ds(NF * 8, 8)], n_sem)

    def out_copy(c, p):
        return pltpu.make_async_copy(
            obuf.at[p], out_hbm.at[pl.ds(blk0 + c * CB, CB)], out_sem)

    def chunk_body(ci, carry):
        p = lax.rem(ci, 2)

        @pl.when(ci >= 2)
        def _():
            out_copy(ci - 2, p).wait()

        for kb in range(CB):
            gather_copy(ci, p, kb).start()
            nemb_copy(ci, p, kb).start()
        for kb in range(CB):
            gather_copy(ci, p, kb).wait()
            nemb_copy(ci, p, kb).wait()

        out_copy(ci, p).start()
        return carry

    lax.fori_loop(0, NCHUNK, chunk_body, 0)

    out_copy(NCHUNK - 2, 0).wait()
    out_copy(NCHUNK - 1, 1).wait()


def kernel(categorical_x, numerical_x, emb_table, W, b):
    cat = categorical_x.astype(jnp.int32)
    catp = cat.reshape(B // 8, 8, NF).transpose(0, 2, 1).reshape(B // 8, NF * 8)
    xp = jnp.pad(numerical_x, ((0, 0), (0, NNP - NN)))
    wt = jnp.pad(W.T, ((0, NNP - NN), (0, 0)))
    nemb = _lin_kernel(xp, wt, b.reshape(1, EMB))
    out4 = _emb_kernel(catp, nemb, emb_table)
    out4 = out4.reshape(B // 8, NR, 8, EMB)
    return out4.transpose(0, 2, 1, 3).reshape(B, NR * EMB)


# speedup vs baseline: 12.6303x; 1.1119x over previous
import functools

import jax
import jax.numpy as jnp
from jax import lax
from jax.experimental import pallas as pl
from jax.experimental.pallas import tpu as pltpu
from jax.experimental.pallas import tpu_sc as plsc

B = 16384
NF = 16
NN = 63
NNP = 64
EMB = 128
NR = NF + 1

_info = plsc.get_sparse_core_info()
NC = _info.num_cores
NS = _info.num_subcores
NW = NC * NS
SPW = B // NW
CH = 16
CB = CH // 8
NCHUNK = SPW // CH

TB = 4096

_mesh = plsc.VectorSubcoreMesh(core_axis_name="c", subcore_axis_name="s")


def _lin_body(x_ref, w_ref, b_ref, o_ref):
    o_ref[...] = (
        lax.dot_general(x_ref[...], w_ref[...], (((1,), (1,)), ((), ())),
                        preferred_element_type=jnp.float32)
        + b_ref[...]
    )


_lin_kernel = pl.pallas_call(
    _lin_body,
    out_shape=jax.ShapeDtypeStruct((B, EMB), jnp.float32),
    grid=(B // TB,),
    in_specs=[
        pl.BlockSpec((TB, NN), lambda i: (i, 0)),
        pl.BlockSpec((EMB, NN), lambda i: (0, 0)),
        pl.BlockSpec((1, EMB), lambda i: (0, 0)),
    ],
    out_specs=pl.BlockSpec((TB, EMB), lambda i: (i, 0)),
)


@functools.partial(
    pl.kernel,
    out_type=jax.ShapeDtypeStruct((B // 8, NR * 8, EMB), jnp.float32),
    mesh=_mesh,
    scratch_types=[
        pltpu.VMEM((SPW // 8, NF * 8), jnp.int32),
        pltpu.VMEM((2, CB, NR * 8, EMB), jnp.float32),
        pltpu.VMEM_SHARED((128, EMB), jnp.float32),
        pltpu.SemaphoreType.DMA,
        pltpu.SemaphoreType.DMA,
        pltpu.SemaphoreType.DMA,
    ],
)
def _emb_kernel(catp_hbm, nemb_hbm, tbl_hbm, out_hbm,
                idx_v, obuf, tbl_sh, g_sem, n_sem, out_sem):
    wid = lax.axis_index("c") * NS + lax.axis_index("s")
    base0 = pl.multiple_of(wid * SPW, 8)
    blk0 = pl.multiple_of(wid * (SPW // 8), 8)

    @pl.when(lax.axis_index("s") == 0)
    def _():
        pltpu.sync_copy(tbl_hbm, tbl_sh.at[pl.ds(0, 127)])

    pltpu.sync_copy(catp_hbm.at[pl.ds(blk0, SPW // 8)], idx_v)
    plsc.subcore_barrier()

    def gather_copy(ci, p, kb):
        return pltpu.make_async_copy(
            tbl_sh.at[idx_v.at[ci * CB + kb]],
            obuf.at[p, kb, pl.ds(0, NF * 8)], g_sem)

    def nemb_copy(ci, p, kb):
        return pltpu.make_async_copy(
            nemb_hbm.at[pl.ds(pl.multiple_of(base0 + ci * CH + kb * 8, 8), 8)],
            obuf.at[p, kb, pl.ds(NF * 8, 8)], n_sem)

    def out_copy(c, p):
        return pltpu.make_async_copy(
            obuf.at[p], out_hbm.at[pl.ds(blk0 + c * CB, CB)], out_sem)

    def chunk_body(ci, carry):
        p = lax.rem(ci, 2)

        @pl.when(ci >= 2)
        def _():
            out_copy(ci - 2, p).wait()

        for kb in range(CB):
            gather_copy(ci, p, kb).start()
            nemb_copy(ci, p, kb).start()
        for kb in range(CB):
            gather_copy(ci, p, kb).wait()
            nemb_copy(ci, p, kb).wait()

        out_copy(ci, p).start()
        return carry

    lax.fori_loop(0, NCHUNK, chunk_body, 0)

    out_copy(NCHUNK - 2, 0).wait()
    out_copy(NCHUNK - 1, 1).wait()


def kernel(categorical_x, numerical_x, emb_table, W, b):
    cat = categorical_x.astype(jnp.int32)
    catp = cat.reshape(B // 8, 8, NF).transpose(0, 2, 1).reshape(B // 8, NF * 8)
    nemb = _lin_kernel(numerical_x, W, b.reshape(1, EMB))
    out4 = _emb_kernel(catp, nemb, emb_table)
    out4 = out4.reshape(B // 8, NR, 8, EMB)
    return out4.transpose(0, 2, 1, 3).reshape(B, NR * EMB)


# speedup vs baseline: 12.7934x; 1.0129x over previous
import functools

import jax
import jax.numpy as jnp
from jax import lax
from jax.experimental import pallas as pl
from jax.experimental.pallas import tpu as pltpu
from jax.experimental.pallas import tpu_sc as plsc

B = 16384
NF = 16
NN = 63
NNP = 64
EMB = 128
NR = NF + 1

_info = plsc.get_sparse_core_info()
NC = _info.num_cores
NS = _info.num_subcores
NW = NC * NS
SPW = B // NW
CH = 16
CB = CH // 8
NCHUNK = SPW // CH

TB = 4096

_mesh = plsc.VectorSubcoreMesh(core_axis_name="c", subcore_axis_name="s")


def _lin_body(x_ref, w_ref, b_ref, o_ref):
    o_ref[...] = (
        lax.dot_general(x_ref[...], w_ref[...], (((1,), (1,)), ((), ())),
                        preferred_element_type=jnp.float32)
        + b_ref[...]
    )


_lin_kernel = pl.pallas_call(
    _lin_body,
    out_shape=jax.ShapeDtypeStruct((B, EMB), jnp.float32),
    grid=(B // TB,),
    in_specs=[
        pl.BlockSpec((TB, NN), lambda i: (i, 0)),
        pl.BlockSpec((EMB, NN), lambda i: (0, 0)),
        pl.BlockSpec((1, EMB), lambda i: (0, 0)),
    ],
    out_specs=pl.BlockSpec((TB, EMB), lambda i: (i, 0)),
)


@functools.partial(
    pl.kernel,
    out_type=jax.ShapeDtypeStruct((B // 8, NR * 8, EMB), jnp.float32),
    mesh=_mesh,
    scratch_types=[
        pltpu.VMEM((SPW // 8, NF * 8), jnp.int32),
        pltpu.VMEM((4, NR * 8, EMB), jnp.float32),
        pltpu.VMEM_SHARED((128, EMB), jnp.float32),
        pltpu.SemaphoreType.DMA,
        pltpu.SemaphoreType.DMA,
        pltpu.SemaphoreType.DMA,
    ],
)
def _emb_kernel(catp_hbm, nemb_hbm, tbl_hbm, out_hbm,
                idx_v, obuf, tbl_sh, g_sem, n_sem, out_sem):
    wid = lax.axis_index("c") * NS + lax.axis_index("s")
    base0 = pl.multiple_of(wid * SPW, 8)
    blk0 = pl.multiple_of(wid * (SPW // 8), 8)

    @pl.when(lax.axis_index("s") == 0)
    def _():
        pltpu.sync_copy(tbl_hbm, tbl_sh.at[pl.ds(0, 127)])

    pltpu.sync_copy(catp_hbm.at[pl.ds(blk0, SPW // 8)], idx_v)
    plsc.subcore_barrier()

    NBLK = SPW // 8

    def gather_copy(blk, p):
        return pltpu.make_async_copy(
            tbl_sh.at[idx_v.at[blk]],
            obuf.at[p, pl.ds(0, NF * 8)], g_sem)

    def nemb_copy(blk, p):
        return pltpu.make_async_copy(
            nemb_hbm.at[pl.ds(pl.multiple_of(base0 + blk * 8, 8), 8)],
            obuf.at[p, pl.ds(NF * 8, 8)], n_sem)

    def out_copy(blk, p):
        return pltpu.make_async_copy(
            obuf.at[p], out_hbm.at[blk0 + blk], out_sem)

    def blk_body(blk, carry):
        p = lax.rem(blk, 4)

        @pl.when(blk >= 4)
        def _():
            out_copy(blk - 4, p).wait()

        gather_copy(blk, p).start()
        nemb_copy(blk, p).start()

        @pl.when(blk >= 1)
        def _():
            pm = lax.rem(blk + 3, 4)
            gather_copy(blk - 1, pm).wait()
            nemb_copy(blk - 1, pm).wait()
            out_copy(blk - 1, pm).start()
        return carry

    lax.fori_loop(0, NBLK, blk_body, 0)

    pl_last = lax.rem(NBLK - 1, 4)
    gather_copy(NBLK - 1, pl_last).wait()
    nemb_copy(NBLK - 1, pl_last).wait()
    out_copy(NBLK - 1, pl_last).start()
    for k in range(4):
        out_copy(NBLK - 4 + k, lax.rem(NBLK - 4 + k, 4)).wait()


def kernel(categorical_x, numerical_x, emb_table, W, b):
    cat = categorical_x.astype(jnp.int32)
    catp = cat.reshape(B // 8, 8, NF).transpose(0, 2, 1).reshape(B // 8, NF * 8)
    nemb = _lin_kernel(numerical_x, W, b.reshape(1, EMB))
    out4 = _emb_kernel(catp, nemb, emb_table)
    out4 = out4.reshape(B // 8, NR, 8, EMB)
    return out4.transpose(0, 2, 1, 3).reshape(B, NR * EMB)
